# Initial kernel scaffold; baseline (speedup 1.0000x reference)
import jax
import jax.numpy as jnp
from jax.experimental import pallas as pl


def kernel(input_ids, token_type_ids, video_embeddings, word_table, ln1_g, ln1_b, W, b, ln2_g, ln2_b, tt_table, ln3_g, ln3_b):
    raise NotImplementedError("write your pallas kernel here")


# speedup vs baseline: 2.3058x; 2.3058x over previous
import functools

import numpy as np
import jax
import jax.numpy as jnp
from jax import lax
from jax.experimental import pallas as pl
from jax.experimental.pallas import tpu as pltpu
from jax.experimental.pallas import tpu_sc as plsc

_VOCAB = 100000
_WVEC = 300
_HID = 768
_MAXV = 100
_MAXT = 25
_B = 1024
_L = _MAXV + _MAXT
_N = _B * _L
_EPS = 1e-12

_NC = 2
_NS = 16
_NW = _NC * _NS
_PER_W = _N // _NW
_CH = 80
_NCH = _PER_W // _CH
_WP = 384


def _pos_encoding():
    pos = np.arange(_L, dtype=np.float32)[:, None]
    div = np.exp(np.arange(0, _HID, 2, dtype=np.float32)
                 * np.float32(-np.log(10000.0) / _HID)).astype(np.float32)
    pe = np.zeros((_L, _HID), np.float32)
    pe[:, 0::2] = np.sin(pos * div)
    pe[:, 1::2] = np.cos(pos * div)
    return jnp.asarray(pe.reshape(_L, 1, _HID))


_POS = _pos_encoding()


def _sc_gather(idx_flat, table):
    mesh = plsc.VectorSubcoreMesh(core_axis_name="c", subcore_axis_name="s")

    @functools.partial(
        pl.kernel,
        mesh=mesh,
        out_type=jax.ShapeDtypeStruct((_N, _WP), jnp.float32),
        scratch_types=[
            pltpu.VMEM((_PER_W,), jnp.int32),
            pltpu.VMEM((_CH, _WP), jnp.float32),
            pltpu.VMEM((_CH, _WP), jnp.float32),
            pltpu.SemaphoreType.DMA,
            pltpu.SemaphoreType.DMA,
        ],
    )
    def k(idx_hbm, table_hbm, out_hbm, idx_v, rows0, rows1, sem0, sem1):
        wid = lax.axis_index("s") * _NC + lax.axis_index("c")
        base = wid * _PER_W
        pltpu.sync_copy(idx_hbm.at[pl.ds(base, _PER_W)], idx_v)
        bufs = (rows0, rows1)
        sems = (sem0, sem1)

        def idx_at(j):
            return idx_v.at[pl.ds(pl.multiple_of(j * _CH, _CH), _CH)]

        pltpu.async_copy(table_hbm.at[idx_at(0)], bufs[0], sems[0])

        def body(i, carry):
            for b2 in range(2):
                j = i * 2 + b2

                @pl.when(j + 1 < _NCH)
                def _():
                    pltpu.async_copy(table_hbm.at[idx_at(j + 1)],
                                     bufs[1 - b2], sems[1 - b2])

                pltpu.make_async_copy(table_hbm.at[idx_at(j)], bufs[b2],
                                      sems[b2]).wait()
                row = pl.multiple_of(base + j * _CH, _CH)
                pltpu.sync_copy(bufs[b2], out_hbm.at[pl.ds(row, _CH)])
            return carry

        lax.fori_loop(0, _NCH // 2, body, 0)

    return k(idx_flat, table)


def _tc_body(we_ref, tti_ref, vid_ref, W_ref, b_ref, g1_ref, b1_ref,
             g2_ref, b2_ref, tt_ref, g3_ref, b3_ref, pos_ref, out_ref):
    l = pl.program_id(0)
    x = we_ref[0][:, :_WVEC]
    u = jnp.mean(x, axis=-1, keepdims=True)
    xc = x - u
    s = jnp.mean(xc * xc, axis=-1, keepdims=True)
    xn = xc * lax.rsqrt(s + _EPS) * g1_ref[0] + b1_ref[0]
    h = jnp.dot(xn, W_ref[...], preferred_element_type=jnp.float32) + b_ref[0]
    h = jnp.maximum(h, 0.0)
    u2 = jnp.mean(h, axis=-1, keepdims=True)
    hc = h - u2
    s2 = jnp.mean(hc * hc, axis=-1, keepdims=True)
    words = hc * lax.rsqrt(s2 + _EPS) * g2_ref[0] + b2_ref[0]
    vmask = jnp.where(l < _MAXV, 1.0, 0.0)
    vid = vid_ref[:, 0, 0, :] * vmask
    m = (tti_ref[0, 0] == 0).astype(jnp.float32)[:, None]
    tt = tt_ref[0] * m + tt_ref[1] * (1.0 - m)
    emb = words + vid + tt + pos_ref[0]
    u3 = jnp.mean(emb, axis=-1, keepdims=True)
    ec = emb - u3
    s3 = jnp.mean(ec * ec, axis=-1, keepdims=True)
    out_ref[:, 0, 0, :] = ec * lax.rsqrt(s3 + _EPS) * g3_ref[0] + b3_ref[0]


_TC_GRID = (_L,)
_TC_IN_SPECS = [
    pl.BlockSpec((1, _B, _WP), lambda l: (l, 0, 0)),
    pl.BlockSpec((1, 1, _B), lambda l: (l, 0, 0)),
    pl.BlockSpec((_B, 1, 1, _HID), lambda l: (0, jnp.minimum(l, _MAXV - 1), 0, 0)),
    pl.BlockSpec((_WVEC, _HID), lambda l: (0, 0)),
    pl.BlockSpec((1, _HID), lambda l: (0, 0)),
    pl.BlockSpec((1, _WVEC), lambda l: (0, 0)),
    pl.BlockSpec((1, _WVEC), lambda l: (0, 0)),
    pl.BlockSpec((1, _HID), lambda l: (0, 0)),
    pl.BlockSpec((1, _HID), lambda l: (0, 0)),
    pl.BlockSpec((2, _HID), lambda l: (0, 0)),
    pl.BlockSpec((1, _HID), lambda l: (0, 0)),
    pl.BlockSpec((1, _HID), lambda l: (0, 0)),
    pl.BlockSpec((1, 1, _HID), lambda l: (l, 0, 0)),
]
_TC_OUT_SPEC = pl.BlockSpec((_B, 1, 1, _HID), lambda l: (0, l, 0, 0))
_TC_OUT_SHAPE = jax.ShapeDtypeStruct((_B, _L, 1, _HID), jnp.float32)


def _tc_fused(*args):
    return pl.pallas_call(
        _tc_body,
        grid=_TC_GRID,
        in_specs=_TC_IN_SPECS,
        out_specs=_TC_OUT_SPEC,
        out_shape=_TC_OUT_SHAPE,
    )(*args)


def kernel(input_ids, token_type_ids, video_embeddings, word_table,
           ln1_g, ln1_b, W, b, ln2_g, ln2_b, tt_table, ln3_g, ln3_b):
    idx_t = input_ids.astype(jnp.int32).T.reshape(-1)
    table_p = jnp.pad(word_table, ((0, 0), (0, _WP - _WVEC)))
    we = _sc_gather(idx_t, table_p)
    we3 = we.reshape(_L, _B, _WP)
    tti3 = token_type_ids.astype(jnp.int32).T.reshape(_L, 1, _B)
    vid4 = video_embeddings.reshape(_B, _MAXV, 1, _HID)
    out4 = _tc_fused(
        we3, tti3, vid4, W, b.reshape(1, _HID),
        ln1_g.reshape(1, _WVEC), ln1_b.reshape(1, _WVEC),
        ln2_g.reshape(1, _HID), ln2_b.reshape(1, _HID),
        tt_table, ln3_g.reshape(1, _HID), ln3_b.reshape(1, _HID),
        _POS)
    return out4.reshape(_B, _L, _HID)


# speedup vs baseline: 2.8647x; 1.2424x over previous
import functools

import numpy as np
import jax
import jax.numpy as jnp
from jax import lax
from jax.experimental import pallas as pl
from jax.experimental.pallas import tpu as pltpu
from jax.experimental.pallas import tpu_sc as plsc

_VOCAB = 100000
_WVEC = 300
_HID = 768
_MAXV = 100
_MAXT = 25
_B = 1024
_L = _MAXV + _MAXT
_N = _B * _L
_EPS = 1e-12

_NC = 2
_NS = 16
_NW = _NC * _NS
_PER_W = _N // _NW
_CH = 80
_NCH = _PER_W // _CH
_WP = 384


def _pos_encoding():
    pos = np.arange(_L, dtype=np.float32)[:, None]
    div = np.exp(np.arange(0, _HID, 2, dtype=np.float32)
                 * np.float32(-np.log(10000.0) / _HID)).astype(np.float32)
    pe = np.zeros((_L, _HID), np.float32)
    pe[:, 0::2] = np.sin(pos * div)
    pe[:, 1::2] = np.cos(pos * div)
    return jnp.asarray(pe.reshape(_L, 1, _HID))


_POS = _pos_encoding()


def _sc_gather(idx_flat, table):
    mesh = plsc.VectorSubcoreMesh(core_axis_name="c", subcore_axis_name="s")

    @functools.partial(
        pl.kernel,
        mesh=mesh,
        out_type=jax.ShapeDtypeStruct((_N, _WP), jnp.float32),
        scratch_types=[
            pltpu.VMEM((_PER_W,), jnp.int32),
            pltpu.VMEM((_CH, _WP), jnp.float32),
            pltpu.VMEM((_CH, _WP), jnp.float32),
            pltpu.SemaphoreType.DMA,
            pltpu.SemaphoreType.DMA,
        ],
    )
    def k(idx_hbm, table_hbm, out_hbm, idx_v, rows0, rows1, sem0, sem1):
        wid = lax.axis_index("s") * _NC + lax.axis_index("c")
        base = wid * _PER_W
        pltpu.sync_copy(idx_hbm.at[pl.ds(base, _PER_W)], idx_v)
        bufs = (rows0, rows1)
        sems = (sem0, sem1)

        def idx_at(j):
            return idx_v.at[pl.ds(pl.multiple_of(j * _CH, _CH), _CH)]

        pltpu.async_copy(table_hbm.at[idx_at(0)], bufs[0], sems[0])

        def body(i, carry):
            for b2 in range(2):
                j = i * 2 + b2

                @pl.when(j + 1 < _NCH)
                def _():
                    pltpu.async_copy(table_hbm.at[idx_at(j + 1)],
                                     bufs[1 - b2], sems[1 - b2])

                pltpu.make_async_copy(table_hbm.at[idx_at(j)], bufs[b2],
                                      sems[b2]).wait()
                row = pl.multiple_of(base + j * _CH, _CH)
                pltpu.sync_copy(bufs[b2], out_hbm.at[pl.ds(row, _CH)])
            return carry

        lax.fori_loop(0, _NCH // 2, body, 0)

    return k(idx_flat, table)


_PAD_ROWS = 4000


def _pad_body(src_ref, dst_ref):
    dst_ref[:, : _WVEC] = src_ref[...]
    dst_ref[:, _WVEC:] = jnp.zeros((_PAD_ROWS, _WP - _WVEC), jnp.float32)


def _pad_table(table):
    return pl.pallas_call(
        _pad_body,
        grid=(_VOCAB // _PAD_ROWS,),
        in_specs=[pl.BlockSpec((_PAD_ROWS, _WVEC), lambda i: (i, 0))],
        out_specs=pl.BlockSpec((_PAD_ROWS, _WP), lambda i: (i, 0)),
        out_shape=jax.ShapeDtypeStruct((_VOCAB, _WP), jnp.float32),
    )(table)


def _tc_body(we_ref, tti_ref, vid_ref, W_ref, b_ref, g1_ref, b1_ref,
             g2_ref, b2_ref, tt_ref, g3_ref, b3_ref, pos_ref, out_ref):
    l = pl.program_id(0)
    x = we_ref[0][:, :_WVEC]
    u = jnp.mean(x, axis=-1, keepdims=True)
    xc = x - u
    s = jnp.mean(xc * xc, axis=-1, keepdims=True)
    xn = xc * lax.rsqrt(s + _EPS) * g1_ref[0] + b1_ref[0]
    h = jnp.dot(xn, W_ref[...], preferred_element_type=jnp.float32) + b_ref[0]
    h = jnp.maximum(h, 0.0)
    u2 = jnp.mean(h, axis=-1, keepdims=True)
    hc = h - u2
    s2 = jnp.mean(hc * hc, axis=-1, keepdims=True)
    words = hc * lax.rsqrt(s2 + _EPS) * g2_ref[0] + b2_ref[0]
    vmask = jnp.where(l < _MAXV, 1.0, 0.0)
    vid = vid_ref[:, 0, 0, :] * vmask
    m = (tti_ref[0, 0] == 0).astype(jnp.float32)[:, None]
    tt = tt_ref[0] * m + tt_ref[1] * (1.0 - m)
    emb = words + vid + tt + pos_ref[0]
    u3 = jnp.mean(emb, axis=-1, keepdims=True)
    ec = emb - u3
    s3 = jnp.mean(ec * ec, axis=-1, keepdims=True)
    out_ref[:, 0, 0, :] = ec * lax.rsqrt(s3 + _EPS) * g3_ref[0] + b3_ref[0]


_TC_GRID = (_L,)
_TC_IN_SPECS = [
    pl.BlockSpec((1, _B, _WP), lambda l: (l, 0, 0)),
    pl.BlockSpec((1, 1, _B), lambda l: (l, 0, 0)),
    pl.BlockSpec((_B, 1, 1, _HID), lambda l: (0, jnp.minimum(l, _MAXV - 1), 0, 0)),
    pl.BlockSpec((_WVEC, _HID), lambda l: (0, 0)),
    pl.BlockSpec((1, _HID), lambda l: (0, 0)),
    pl.BlockSpec((1, _WVEC), lambda l: (0, 0)),
    pl.BlockSpec((1, _WVEC), lambda l: (0, 0)),
    pl.BlockSpec((1, _HID), lambda l: (0, 0)),
    pl.BlockSpec((1, _HID), lambda l: (0, 0)),
    pl.BlockSpec((2, _HID), lambda l: (0, 0)),
    pl.BlockSpec((1, _HID), lambda l: (0, 0)),
    pl.BlockSpec((1, _HID), lambda l: (0, 0)),
    pl.BlockSpec((1, 1, _HID), lambda l: (l, 0, 0)),
]
_TC_OUT_SPEC = pl.BlockSpec((_B, 1, 1, _HID), lambda l: (0, l, 0, 0))
_TC_OUT_SHAPE = jax.ShapeDtypeStruct((_B, _L, 1, _HID), jnp.float32)


def _tc_fused(*args):
    return pl.pallas_call(
        _tc_body,
        grid=_TC_GRID,
        in_specs=_TC_IN_SPECS,
        out_specs=_TC_OUT_SPEC,
        out_shape=_TC_OUT_SHAPE,
    )(*args)


def kernel(input_ids, token_type_ids, video_embeddings, word_table,
           ln1_g, ln1_b, W, b, ln2_g, ln2_b, tt_table, ln3_g, ln3_b):
    idx_t = input_ids.astype(jnp.int32).T.reshape(-1)
    table_p = _pad_table(word_table)
    we = _sc_gather(idx_t, table_p)
    we3 = we.reshape(_L, _B, _WP)
    tti3 = token_type_ids.astype(jnp.int32).T.reshape(_L, 1, _B)
    vid4 = video_embeddings.reshape(_B, _MAXV, 1, _HID)
    out4 = _tc_fused(
        we3, tti3, vid4, W, b.reshape(1, _HID),
        ln1_g.reshape(1, _WVEC), ln1_b.reshape(1, _WVEC),
        ln2_g.reshape(1, _HID), ln2_b.reshape(1, _HID),
        tt_table, ln3_g.reshape(1, _HID), ln3_b.reshape(1, _HID),
        _POS)
    return out4.reshape(_B, _L, _HID)
